# Initial kernel scaffold; baseline (speedup 1.0000x reference)
import functools

import jax
import jax.numpy as jnp
from jax import lax
from jax.experimental import pallas as pl
from jax.experimental.pallas import tpu as pltpu, tpu_sc as plsc

N = 10000
E = 320000
D = 128
G = 64
C = 125
EC = E // C
NC = N // C
CHUNKS_PER_TILE = EC // 32
ROWS_PER_TILE = N // 16

_MESH = plsc.VectorSubcoreMesh(core_axis_name="c", subcore_axis_name="s")


@functools.partial(
    pl.kernel,
    out_type=(
        jax.ShapeDtypeStruct((2 * N, 16), jnp.float32),
        jax.ShapeDtypeStruct((G, 16), jnp.float32),
    ),
    mesh=_MESH,
    scratch_types=[
        pltpu.VMEM((CHUNKS_PER_TILE, C), jnp.int32),
        pltpu.VMEM((5, C), jnp.int32),
        pltpu.VMEM((C, 16), jnp.float32),
        pltpu.VMEM((C, 16), jnp.float32),
        pltpu.VMEM_SHARED((N, 16), jnp.float32),
        pltpu.VMEM_SHARED((G, 16), jnp.float32),
    ],
)
def _sc_precompute(dst2d, batch2d, deg_out, cnt_out,
                   dbuf, bbuf, ones, zbuf, deg_sh, cnt_sh):
    c = lax.axis_index("c")
    s = lax.axis_index("s")
    row0 = s * ROWS_PER_TILE

    def fill(i, _):
        ones[i, :] = jnp.ones((16,), jnp.float32)
        zbuf[i, :] = jnp.zeros((16,), jnp.float32)
        return 0
    lax.fori_loop(0, C, fill, 0)

    for k in range(5):
        pltpu.sync_copy(zbuf, deg_sh.at[pl.ds(row0 + k * C, C)])

    @pl.when(jnp.logical_and(c == 0, s == 0))
    def _():
        pltpu.sync_copy(zbuf.at[pl.ds(0, G)], cnt_sh)

    base = c * (EC // 2) + s * CHUNKS_PER_TILE
    pltpu.sync_copy(dst2d.at[pl.ds(base, CHUNKS_PER_TILE)], dbuf)

    @pl.when(c == 0)
    def _():
        for k in range(5):
            pltpu.sync_copy(batch2d.at[s + 16 * k], bbuf.at[k])

    plsc.subcore_barrier()

    def edge_body(j, _):
        pltpu.sync_copy(ones, deg_sh.at[dbuf.at[j]], add=True)
        return 0
    lax.fori_loop(0, CHUNKS_PER_TILE, edge_body, 0)

    @pl.when(c == 0)
    def _():
        for k in range(5):
            pltpu.sync_copy(ones, cnt_sh.at[bbuf.at[k]], add=True)

    plsc.subcore_barrier()

    for k in range(5):
        sl = pl.ds(row0 + k * C, C)
        pltpu.sync_copy(deg_sh.at[sl], zbuf)
        pltpu.sync_copy(zbuf, deg_out.at[pl.ds(c * N + row0 + k * C, C)])

    @pl.when(jnp.logical_and(c == 0, s == 0))
    def _():
        pltpu.sync_copy(cnt_sh, zbuf.at[pl.ds(0, G)])
        pltpu.sync_copy(zbuf.at[pl.ds(0, G)], cnt_out)


@functools.partial(
    pl.kernel,
    out_type=jax.ShapeDtypeStruct((2 * N, D), jnp.float32),
    mesh=_MESH,
    scratch_types=[
        pltpu.VMEM((CHUNKS_PER_TILE, C), jnp.int32),
        pltpu.VMEM((CHUNKS_PER_TILE, C), jnp.int32),
        pltpu.VMEM((C, D), jnp.float32),
        pltpu.VMEM_SHARED((N, D), jnp.float32),
    ],
)
def _sc_edge(y, src2d, dst2d, zeros, s_out, isrc, idst, rbuf, s_sh):
    c = lax.axis_index("c")
    s = lax.axis_index("s")
    row0 = s * ROWS_PER_TILE

    for k in range(5):
        sl = pl.ds(row0 + k * C, C)

        @pl.when(c == 0)
        def _():
            pltpu.sync_copy(y.at[sl], rbuf)
            pltpu.sync_copy(rbuf, s_sh.at[sl])

        @pl.when(c == 1)
        def _():
            pltpu.sync_copy(zeros.at[sl], rbuf)
            pltpu.sync_copy(rbuf, s_sh.at[sl])

    base = c * (EC // 2) + s * CHUNKS_PER_TILE
    pltpu.sync_copy(src2d.at[pl.ds(base, CHUNKS_PER_TILE)], isrc)
    pltpu.sync_copy(dst2d.at[pl.ds(base, CHUNKS_PER_TILE)], idst)

    plsc.subcore_barrier()

    def edge_body(j, _):
        pltpu.sync_copy(y.at[isrc.at[j]], rbuf)
        pltpu.sync_copy(rbuf, s_sh.at[idst.at[j]], add=True)
        return 0
    lax.fori_loop(0, CHUNKS_PER_TILE, edge_body, 0)

    plsc.subcore_barrier()

    for k in range(5):
        sl = pl.ds(row0 + k * C, C)
        pltpu.sync_copy(s_sh.at[sl], rbuf)
        pltpu.sync_copy(rbuf, s_out.at[pl.ds(c * N + row0 + k * C, C)])


@functools.partial(
    pl.kernel,
    out_type=jax.ShapeDtypeStruct((2 * G, D), jnp.float32),
    mesh=_MESH,
    scratch_types=[
        pltpu.VMEM((3, C), jnp.int32),
        pltpu.VMEM((C, D), jnp.float32),
        pltpu.VMEM_SHARED((G, D), jnp.float32),
    ],
)
def _sc_pool(z, batch2d, zeros, p_out, bbuf, rbuf, p_sh):
    c = lax.axis_index("c")
    s = lax.axis_index("s")
    wid = c * 16 + s

    @pl.when(s == 0)
    def _():
        pltpu.sync_copy(zeros.at[pl.ds(0, G)], rbuf.at[pl.ds(0, G)])
        pltpu.sync_copy(rbuf.at[pl.ds(0, G)], p_sh)

    for k in range(3):
        ch = wid + 32 * k

        @pl.when(ch < NC)
        def _():
            pltpu.sync_copy(batch2d.at[ch], bbuf.at[k])

    plsc.subcore_barrier()

    for k in range(3):
        ch = wid + 32 * k

        @pl.when(ch < NC)
        def _():
            pltpu.sync_copy(z.at[pl.ds(ch * C, C)], rbuf)
            pltpu.sync_copy(rbuf, p_sh.at[bbuf.at[k]], add=True)

    plsc.subcore_barrier()

    @pl.when(s == 0)
    def _():
        pltpu.sync_copy(p_sh, rbuf.at[pl.ds(0, G)])
        pltpu.sync_copy(rbuf.at[pl.ds(0, G)], p_out.at[pl.ds(c * G, G)])


R = 1000


def _dinv_body(deg_ref, out_ref):
    deg = deg_ref[0][:, :1] + deg_ref[1][:, :1] + 1.0
    out_ref[...] = lax.rsqrt(deg)


def _tc_dinv(deg16):
    return pl.pallas_call(
        _dinv_body,
        grid=(N // R,),
        in_specs=[pl.BlockSpec((2, R, 16), lambda r: (0, r, 0))],
        out_specs=pl.BlockSpec((R, 1), lambda r: (r, 0)),
        out_shape=jax.ShapeDtypeStruct((N, 1), jnp.float32),
    )(deg16)


def _mm1_body(x_ref, w_ref, dinv_ref, out_ref):
    out_ref[...] = dinv_ref[...] * jnp.dot(
        x_ref[...], w_ref[...], preferred_element_type=jnp.float32)


def _tc_mm1(x, w, dinv):
    return pl.pallas_call(
        _mm1_body,
        grid=(N // R,),
        in_specs=[
            pl.BlockSpec((R, D), lambda r: (r, 0)),
            pl.BlockSpec((D, D), lambda r: (0, 0)),
            pl.BlockSpec((R, 1), lambda r: (r, 0)),
        ],
        out_specs=pl.BlockSpec((R, D), lambda r: (r, 0)),
        out_shape=jax.ShapeDtypeStruct((N, D), jnp.float32),
    )(x, w, dinv)


def _layer_body(sp_ref, dinv_ref, b_ref, w_ref, out_ref):
    h = sp_ref[0] + sp_ref[1]
    h = jnp.maximum(dinv_ref[...] * h + b_ref[...], 0.0)
    out_ref[...] = dinv_ref[...] * jnp.dot(
        h, w_ref[...], preferred_element_type=jnp.float32)


def _tc_layer(s_part, dinv, b, w):
    return pl.pallas_call(
        _layer_body,
        grid=(N // R,),
        in_specs=[
            pl.BlockSpec((2, R, D), lambda r: (0, r, 0)),
            pl.BlockSpec((R, 1), lambda r: (r, 0)),
            pl.BlockSpec((1, D), lambda r: (0, 0)),
            pl.BlockSpec((D, D), lambda r: (0, 0)),
        ],
        out_specs=pl.BlockSpec((R, D), lambda r: (r, 0)),
        out_shape=jax.ShapeDtypeStruct((N, D), jnp.float32),
    )(s_part, dinv, b, w)


def _z_body(sp_ref, dinv_ref, b_ref, out_ref):
    out_ref[...] = dinv_ref[...] * (sp_ref[0] + sp_ref[1]) + b_ref[...]


def _tc_z(s_part, dinv, b):
    return pl.pallas_call(
        _z_body,
        grid=(N // R,),
        in_specs=[
            pl.BlockSpec((2, R, D), lambda r: (0, r, 0)),
            pl.BlockSpec((R, 1), lambda r: (r, 0)),
            pl.BlockSpec((1, D), lambda r: (0, 0)),
        ],
        out_specs=pl.BlockSpec((R, D), lambda r: (r, 0)),
        out_shape=jax.ShapeDtypeStruct((N, D), jnp.float32),
    )(s_part, dinv, b)


def _final_body(pp_ref, cnt_ref, wl_ref, bl_ref, out_ref):
    pooled = (pp_ref[0] + pp_ref[1]) / jnp.maximum(cnt_ref[:, :1], 1.0)
    out_ref[...] = jnp.dot(
        pooled, wl_ref[...], preferred_element_type=jnp.float32) + bl_ref[...]


def _tc_final(pool_part, cnt16, wlin, blin):
    return pl.pallas_call(
        _final_body,
        out_shape=jax.ShapeDtypeStruct((G, wlin.shape[1]), jnp.float32),
    )(pool_part, cnt16, wlin, blin)


def kernel(x, edge_index, batch, W1, b1, W2, b2, W3, b3, Wlin, blin):
    src2d = edge_index[0].reshape(EC, C)
    dst2d = edge_index[1].reshape(EC, C)
    batch2d = batch.reshape(NC, C)
    zeros = jnp.zeros((N, D), jnp.float32)

    deg16, cnt16 = _sc_precompute(dst2d, batch2d)
    dinv = _tc_dinv(deg16.reshape(2, N, 16))

    y1 = _tc_mm1(x, W1, dinv)
    s1 = _sc_edge(y1, src2d, dst2d, zeros).reshape(2, N, D)
    y2 = _tc_layer(s1, dinv, b1.reshape(1, D), W2)
    s2 = _sc_edge(y2, src2d, dst2d, zeros).reshape(2, N, D)
    y3 = _tc_layer(s2, dinv, b2.reshape(1, D), W3)
    s3 = _sc_edge(y3, src2d, dst2d, zeros).reshape(2, N, D)

    z = _tc_z(s3, dinv, b3.reshape(1, D))
    pool_part = _sc_pool(z, batch2d, zeros).reshape(2, G, D)
    return _tc_final(pool_part, cnt16, Wlin, blin.reshape(1, blin.shape[0]))


# speedup vs baseline: 16.7110x; 16.7110x over previous
import functools

import jax
import jax.numpy as jnp
from jax import lax
from jax.experimental import pallas as pl
from jax.experimental.pallas import tpu as pltpu, tpu_sc as plsc

N = 10000
E = 320000
D = 128
G = 64
C = 125
EC = E // C
CHUNKS_PER_TILE = EC // 32

PC = 80
NPC = N // PC

_MESH = plsc.VectorSubcoreMesh(core_axis_name="c", subcore_axis_name="s")


@functools.partial(
    pl.kernel,
    out_type=(
        jax.ShapeDtypeStruct((2 * N, D), jnp.float32),
        jax.ShapeDtypeStruct((G, D), jnp.float32),
    ),
    mesh=_MESH,
    scratch_types=[
        pltpu.VMEM((CHUNKS_PER_TILE, C), jnp.int32),
        pltpu.VMEM((8, PC), jnp.int32),
        pltpu.VMEM((C, D), jnp.float32),
        pltpu.VMEM((PC, D), jnp.float32),
        pltpu.VMEM_SHARED((N, D), jnp.float32),
        pltpu.VMEM_SHARED((G, D), jnp.float32),
    ],
)
def _sc_precompute(dst2d, batch2d, ones_in, zeros, deg_out, cnt_out,
                   dbuf, bbuf, ones, zbuf, deg_sh, cnt_sh):
    c = lax.axis_index("c")
    s = lax.axis_index("s")

    pltpu.sync_copy(ones_in, ones)
    pltpu.sync_copy(zeros.at[pl.ds(0, PC)], zbuf)

    for k in range(8):
        ch = s + 16 * k

        @pl.when(ch < NPC)
        def _():
            pltpu.sync_copy(zbuf, deg_sh.at[pl.ds(ch * PC, PC)])

    @pl.when(jnp.logical_and(c == 0, s == 0))
    def _():
        pltpu.sync_copy(zbuf.at[pl.ds(0, G)], cnt_sh)

    base = c * (EC // 2) + s * CHUNKS_PER_TILE
    pltpu.sync_copy(dst2d.at[pl.ds(base, CHUNKS_PER_TILE)], dbuf)

    @pl.when(c == 0)
    def _():
        for k in range(8):
            ch = s + 16 * k

            @pl.when(ch < NPC)
            def _():
                pltpu.sync_copy(batch2d.at[ch], bbuf.at[k])

    plsc.subcore_barrier()

    def edge_body(j, _):
        pltpu.sync_copy(ones, deg_sh.at[dbuf.at[j]], add=True)
        return 0
    lax.fori_loop(0, CHUNKS_PER_TILE, edge_body, 0)

    @pl.when(c == 0)
    def _():
        for k in range(8):
            ch = s + 16 * k

            @pl.when(ch < NPC)
            def _():
                pltpu.sync_copy(ones.at[pl.ds(0, PC)],
                                cnt_sh.at[bbuf.at[k]], add=True)

    plsc.subcore_barrier()

    for k in range(8):
        ch = s + 16 * k

        @pl.when(ch < NPC)
        def _():
            sl = pl.ds(ch * PC, PC)
            pltpu.sync_copy(deg_sh.at[sl], zbuf)
            pltpu.sync_copy(zbuf, deg_out.at[pl.ds(c * N + ch * PC, PC)])

    @pl.when(jnp.logical_and(c == 0, s == 0))
    def _():
        pltpu.sync_copy(cnt_sh, zbuf.at[pl.ds(0, G)])
        pltpu.sync_copy(zbuf.at[pl.ds(0, G)], cnt_out)


@functools.partial(
    pl.kernel,
    out_type=jax.ShapeDtypeStruct((2 * N, D), jnp.float32),
    mesh=_MESH,
    scratch_types=[
        pltpu.VMEM((CHUNKS_PER_TILE, C), jnp.int32),
        pltpu.VMEM((CHUNKS_PER_TILE, C), jnp.int32),
        pltpu.VMEM((C, D), jnp.float32),
        pltpu.VMEM_SHARED((N, D), jnp.float32),
    ],
)
def _sc_edge(y, src2d, dst2d, zeros, s_out, isrc, idst, rbuf, s_sh):
    c = lax.axis_index("c")
    s = lax.axis_index("s")
    stage = rbuf.at[pl.ds(0, PC)]

    for k in range(8):
        ch = s + 16 * k

        @pl.when(ch < NPC)
        def _():
            sl = pl.ds(ch * PC, PC)

            @pl.when(c == 0)
            def _():
                pltpu.sync_copy(y.at[sl], stage)

            @pl.when(c == 1)
            def _():
                pltpu.sync_copy(zeros.at[sl], stage)

            pltpu.sync_copy(stage, s_sh.at[sl])

    base = c * (EC // 2) + s * CHUNKS_PER_TILE
    pltpu.sync_copy(src2d.at[pl.ds(base, CHUNKS_PER_TILE)], isrc)
    pltpu.sync_copy(dst2d.at[pl.ds(base, CHUNKS_PER_TILE)], idst)

    plsc.subcore_barrier()

    def edge_body(j, _):
        pltpu.sync_copy(y.at[isrc.at[j]], rbuf)
        pltpu.sync_copy(rbuf, s_sh.at[idst.at[j]], add=True)
        return 0
    lax.fori_loop(0, CHUNKS_PER_TILE, edge_body, 0)

    plsc.subcore_barrier()

    for k in range(8):
        ch = s + 16 * k

        @pl.when(ch < NPC)
        def _():
            sl = pl.ds(ch * PC, PC)
            pltpu.sync_copy(s_sh.at[sl], stage)
            pltpu.sync_copy(stage, s_out.at[pl.ds(c * N + ch * PC, PC)])


@functools.partial(
    pl.kernel,
    out_type=jax.ShapeDtypeStruct((2 * G, D), jnp.float32),
    mesh=_MESH,
    scratch_types=[
        pltpu.VMEM((4, PC), jnp.int32),
        pltpu.VMEM((PC, D), jnp.float32),
        pltpu.VMEM_SHARED((G, D), jnp.float32),
    ],
)
def _sc_pool(z, batch2d, zeros, p_out, bbuf, rbuf, p_sh):
    c = lax.axis_index("c")
    s = lax.axis_index("s")
    wid = c * 16 + s

    @pl.when(s == 0)
    def _():
        pltpu.sync_copy(zeros.at[pl.ds(0, G)], rbuf.at[pl.ds(0, G)])
        pltpu.sync_copy(rbuf.at[pl.ds(0, G)], p_sh)

    for k in range(4):
        ch = wid + 32 * k

        @pl.when(ch < NPC)
        def _():
            pltpu.sync_copy(batch2d.at[ch], bbuf.at[k])

    plsc.subcore_barrier()

    for k in range(4):
        ch = wid + 32 * k

        @pl.when(ch < NPC)
        def _():
            pltpu.sync_copy(z.at[pl.ds(ch * PC, PC)], rbuf)
            pltpu.sync_copy(rbuf, p_sh.at[bbuf.at[k]], add=True)

    plsc.subcore_barrier()

    @pl.when(s == 0)
    def _():
        pltpu.sync_copy(p_sh, rbuf.at[pl.ds(0, G)])
        pltpu.sync_copy(rbuf.at[pl.ds(0, G)], p_out.at[pl.ds(c * G, G)])


R = 1000


def _dinv_body(deg_ref, out_ref):
    deg = deg_ref[0][:, :1] + deg_ref[1][:, :1] + 1.0
    out_ref[...] = lax.rsqrt(deg)


def _tc_dinv(deg2):
    return pl.pallas_call(
        _dinv_body,
        grid=(N // R,),
        in_specs=[pl.BlockSpec((2, R, D), lambda r: (0, r, 0))],
        out_specs=pl.BlockSpec((R, 1), lambda r: (r, 0)),
        out_shape=jax.ShapeDtypeStruct((N, 1), jnp.float32),
    )(deg2)


def _mm1_body(x_ref, w_ref, dinv_ref, out_ref):
    out_ref[...] = dinv_ref[...] * jnp.dot(
        x_ref[...], w_ref[...], preferred_element_type=jnp.float32)


def _tc_mm1(x, w, dinv):
    return pl.pallas_call(
        _mm1_body,
        grid=(N // R,),
        in_specs=[
            pl.BlockSpec((R, D), lambda r: (r, 0)),
            pl.BlockSpec((D, D), lambda r: (0, 0)),
            pl.BlockSpec((R, 1), lambda r: (r, 0)),
        ],
        out_specs=pl.BlockSpec((R, D), lambda r: (r, 0)),
        out_shape=jax.ShapeDtypeStruct((N, D), jnp.float32),
    )(x, w, dinv)


def _layer_body(sp_ref, dinv_ref, b_ref, w_ref, out_ref):
    h = sp_ref[0] + sp_ref[1]
    h = jnp.maximum(dinv_ref[...] * h + b_ref[...], 0.0)
    out_ref[...] = dinv_ref[...] * jnp.dot(
        h, w_ref[...], preferred_element_type=jnp.float32)


def _tc_layer(s_part, dinv, b, w):
    return pl.pallas_call(
        _layer_body,
        grid=(N // R,),
        in_specs=[
            pl.BlockSpec((2, R, D), lambda r: (0, r, 0)),
            pl.BlockSpec((R, 1), lambda r: (r, 0)),
            pl.BlockSpec((1, D), lambda r: (0, 0)),
            pl.BlockSpec((D, D), lambda r: (0, 0)),
        ],
        out_specs=pl.BlockSpec((R, D), lambda r: (r, 0)),
        out_shape=jax.ShapeDtypeStruct((N, D), jnp.float32),
    )(s_part, dinv, b, w)


def _z_body(sp_ref, dinv_ref, b_ref, out_ref):
    out_ref[...] = dinv_ref[...] * (sp_ref[0] + sp_ref[1]) + b_ref[...]


def _tc_z(s_part, dinv, b):
    return pl.pallas_call(
        _z_body,
        grid=(N // R,),
        in_specs=[
            pl.BlockSpec((2, R, D), lambda r: (0, r, 0)),
            pl.BlockSpec((R, 1), lambda r: (r, 0)),
            pl.BlockSpec((1, D), lambda r: (0, 0)),
        ],
        out_specs=pl.BlockSpec((R, D), lambda r: (r, 0)),
        out_shape=jax.ShapeDtypeStruct((N, D), jnp.float32),
    )(s_part, dinv, b)


def _final_body(pp_ref, cnt_ref, wl_ref, bl_ref, out_ref):
    pooled = (pp_ref[0] + pp_ref[1]) / jnp.maximum(cnt_ref[:, :1], 1.0)
    out_ref[...] = jnp.dot(
        pooled, wl_ref[...], preferred_element_type=jnp.float32) + bl_ref[...]


def _tc_final(pool_part, cnt16, wlin, blin):
    return pl.pallas_call(
        _final_body,
        out_shape=jax.ShapeDtypeStruct((G, wlin.shape[1]), jnp.float32),
    )(pool_part, cnt16, wlin, blin)


def kernel(x, edge_index, batch, W1, b1, W2, b2, W3, b3, Wlin, blin):
    src2d = edge_index[0].reshape(EC, C)
    dst2d = edge_index[1].reshape(EC, C)
    batch2d = batch.reshape(NPC, PC)
    zeros = jnp.zeros((N, D), jnp.float32)

    ones_in = jnp.ones((C, D), jnp.float32)
    deg2, cnt = _sc_precompute(dst2d, batch2d, ones_in, zeros)
    dinv = _tc_dinv(deg2.reshape(2, N, D))

    y1 = _tc_mm1(x, W1, dinv)
    s1 = _sc_edge(y1, src2d, dst2d, zeros).reshape(2, N, D)
    y2 = _tc_layer(s1, dinv, b1.reshape(1, D), W2)
    s2 = _sc_edge(y2, src2d, dst2d, zeros).reshape(2, N, D)
    y3 = _tc_layer(s2, dinv, b2.reshape(1, D), W3)
    s3 = _sc_edge(y3, src2d, dst2d, zeros).reshape(2, N, D)

    z = _tc_z(s3, dinv, b3.reshape(1, D))
    pool_part = _sc_pool(z, batch2d, zeros).reshape(2, G, D)
    return _tc_final(pool_part, cnt, Wlin, blin.reshape(1, blin.shape[0]))


# speedup vs baseline: 20.0831x; 1.2018x over previous
import functools

import jax
import jax.numpy as jnp
from jax import lax
from jax.experimental import pallas as pl
from jax.experimental.pallas import tpu as pltpu, tpu_sc as plsc

N = 10000
E = 320000
D = 128
G = 64
PC = 80
NPC = N // PC

_MESH = plsc.VectorSubcoreMesh(core_axis_name="c", subcore_axis_name="s")


@functools.partial(
    pl.kernel,
    out_type=(
        jax.ShapeDtypeStruct((2 * N, D), jnp.float32),
        jax.ShapeDtypeStruct((G, D), jnp.float32),
    ),
    mesh=_MESH,
    scratch_types=[
        pltpu.VMEM((200, 50), jnp.int32),
        pltpu.VMEM((8, PC), jnp.int32),
        pltpu.VMEM((PC, D), jnp.float32),
        pltpu.VMEM((PC, D), jnp.float32),
        pltpu.VMEM_SHARED((N, D), jnp.float32),
        pltpu.VMEM_SHARED((G, D), jnp.float32),
    ],
)
def _sc_precompute(dst2d, batch2d, ones_in, zeros, deg_out, cnt_out,
                   dbuf, bbuf, ones, zbuf, deg_sh, cnt_sh):
    c = lax.axis_index("c")
    s = lax.axis_index("s")

    pltpu.sync_copy(ones_in, ones)
    pltpu.sync_copy(zeros.at[pl.ds(0, PC)], zbuf)

    for k in range(8):
        ch = s + 16 * k

        @pl.when(ch < NPC)
        def _():
            pltpu.sync_copy(zbuf, deg_sh.at[pl.ds(ch * PC, PC)])

    @pl.when(jnp.logical_and(c == 0, s == 0))
    def _():
        pltpu.sync_copy(zbuf.at[pl.ds(0, G)], cnt_sh)

    base = c * 3200 + s * 200
    pltpu.sync_copy(dst2d.at[pl.ds(base, 200)], dbuf)

    @pl.when(c == 0)
    def _():
        for k in range(8):
            ch = s + 16 * k

            @pl.when(ch < NPC)
            def _():
                pltpu.sync_copy(batch2d.at[ch], bbuf.at[k])

    plsc.subcore_barrier()

    def edge_body(j, _):
        pltpu.sync_copy(ones.at[pl.ds(0, 50)], deg_sh.at[dbuf.at[j]], add=True)
        return 0
    lax.fori_loop(0, 200, edge_body, 0)

    @pl.when(c == 0)
    def _():
        for k in range(8):
            ch = s + 16 * k

            @pl.when(ch < NPC)
            def _():
                pltpu.sync_copy(ones.at[pl.ds(0, PC)],
                                cnt_sh.at[bbuf.at[k]], add=True)

    plsc.subcore_barrier()

    for k in range(8):
        ch = s + 16 * k

        @pl.when(ch < NPC)
        def _():
            sl = pl.ds(ch * PC, PC)
            pltpu.sync_copy(deg_sh.at[sl], zbuf)
            pltpu.sync_copy(zbuf, deg_out.at[pl.ds(c * N + ch * PC, PC)])

    @pl.when(jnp.logical_and(c == 0, s == 0))
    def _():
        pltpu.sync_copy(cnt_sh, zbuf.at[pl.ds(0, G)])
        pltpu.sync_copy(zbuf.at[pl.ds(0, G)], cnt_out)


CE = 128
CPT = 79
JUNK_ROWS = 3584


@functools.partial(
    pl.kernel,
    out_type=jax.ShapeDtypeStruct((2 * N, D), jnp.float32),
    mesh=_MESH,
    scratch_types=[
        pltpu.VMEM((80, CE), jnp.int32),
        pltpu.VMEM((2, CE), jnp.int32),
        pltpu.VMEM((2, CE), jnp.int32),
        pltpu.VMEM((CE, D), jnp.float32),
        pltpu.VMEM((CE, D), jnp.float32),
        pltpu.SemaphoreType.DMA,
        pltpu.SemaphoreType.DMA,
        pltpu.VMEM_SHARED((N, D), jnp.float32),
    ],
)
def _sc_edge(y, pk2d, zeros, s_out,
             ipack, sidx, didx, rb0, rb1, gsem, ssem, s_sh):
    c = lax.axis_index("c")
    s = lax.axis_index("s")
    wid = c * 16 + s
    rbs = [rb0, rb1]
    stage = rb0.at[pl.ds(0, PC)]

    for k in range(8):
        ch = s + 16 * k

        @pl.when(ch < NPC)
        def _():
            sl = pl.ds(ch * PC, PC)

            @pl.when(c == 0)
            def _():
                pltpu.sync_copy(y.at[sl], stage)

            @pl.when(c == 1)
            def _():
                pltpu.sync_copy(zeros.at[sl], stage)

            pltpu.sync_copy(stage, s_sh.at[sl])

    pltpu.sync_copy(pk2d.at[pl.ds(wid * 80, 80)], ipack)

    plsc.subcore_barrier()

    def unpack(j, slot):
        for v in range(8):
            pk = ipack[j, pl.ds(16 * v, 16)]
            sidx[slot, pl.ds(16 * v, 16)] = jnp.bitwise_and(pk, 0xFFFF)
            didx[slot, pl.ds(16 * v, 16)] = lax.shift_right_logical(pk, 16)

    unpack(0, 0)
    pltpu.async_copy(y.at[sidx.at[0]], rb0, gsem)

    def edge_body(i, _):
        for b in range(2):
            j = 2 * i + b
            rb, ro = rbs[b], rbs[1 - b]
            pltpu.make_async_copy(y.at[sidx.at[b]], rb, gsem).wait()
            pltpu.async_copy(rb, s_sh.at[didx.at[b]], ssem, add=True)

            @pl.when(j >= 1)
            def _():
                pltpu.make_async_copy(ro, s_sh.at[didx.at[1 - b]], ssem).wait()

            unpack(j + 1, 1 - b)
            pltpu.async_copy(y.at[sidx.at[1 - b]], ro, gsem)
        return 0
    lax.fori_loop(0, (CPT - 1) // 2, edge_body, 0)

    pltpu.make_async_copy(y.at[sidx.at[0]], rb0, gsem).wait()
    pltpu.async_copy(rb0, s_sh.at[didx.at[0]], ssem, add=True)
    pltpu.make_async_copy(rb1, s_sh.at[didx.at[1]], ssem).wait()
    pltpu.make_async_copy(rb0, s_sh.at[didx.at[0]], ssem).wait()

    plsc.subcore_barrier()

    for k in range(8):
        ch = s + 16 * k

        @pl.when(ch < NPC)
        def _():
            sl = pl.ds(ch * PC, PC)
            pltpu.sync_copy(s_sh.at[sl], stage)
            pltpu.sync_copy(stage, s_out.at[pl.ds(c * N + ch * PC, PC)])


@functools.partial(
    pl.kernel,
    out_type=jax.ShapeDtypeStruct((2 * G, D), jnp.float32),
    mesh=_MESH,
    scratch_types=[
        pltpu.VMEM((4, PC), jnp.int32),
        pltpu.VMEM((PC, D), jnp.float32),
        pltpu.VMEM_SHARED((G, D), jnp.float32),
    ],
)
def _sc_pool(z, batch2d, zeros, p_out, bbuf, rbuf, p_sh):
    c = lax.axis_index("c")
    s = lax.axis_index("s")
    wid = c * 16 + s

    @pl.when(s == 0)
    def _():
        pltpu.sync_copy(zeros.at[pl.ds(0, G)], rbuf.at[pl.ds(0, G)])
        pltpu.sync_copy(rbuf.at[pl.ds(0, G)], p_sh)

    for k in range(4):
        ch = wid + 32 * k

        @pl.when(ch < NPC)
        def _():
            pltpu.sync_copy(batch2d.at[ch], bbuf.at[k])

    plsc.subcore_barrier()

    for k in range(4):
        ch = wid + 32 * k

        @pl.when(ch < NPC)
        def _():
            pltpu.sync_copy(z.at[pl.ds(ch * PC, PC)], rbuf)
            pltpu.sync_copy(rbuf, p_sh.at[bbuf.at[k]], add=True)

    plsc.subcore_barrier()

    @pl.when(s == 0)
    def _():
        pltpu.sync_copy(p_sh, rbuf.at[pl.ds(0, G)])
        pltpu.sync_copy(rbuf.at[pl.ds(0, G)], p_out.at[pl.ds(c * G, G)])


R = 1000


def _dinv_body(deg_ref, out_ref):
    deg = deg_ref[0][:, :1] + deg_ref[1][:, :1] + 1.0
    out_ref[...] = lax.rsqrt(deg)


def _tc_dinv(deg2):
    return pl.pallas_call(
        _dinv_body,
        grid=(N // R,),
        in_specs=[pl.BlockSpec((2, R, D), lambda r: (0, r, 0))],
        out_specs=pl.BlockSpec((R, 1), lambda r: (r, 0)),
        out_shape=jax.ShapeDtypeStruct((N, 1), jnp.float32),
    )(deg2)


def _mm1_body(x_ref, w_ref, dinv_ref, out_ref):
    out_ref[...] = dinv_ref[...] * jnp.dot(
        x_ref[...], w_ref[...], preferred_element_type=jnp.float32)


def _tc_mm1(x, w, dinv):
    return pl.pallas_call(
        _mm1_body,
        grid=(N // R,),
        in_specs=[
            pl.BlockSpec((R, D), lambda r: (r, 0)),
            pl.BlockSpec((D, D), lambda r: (0, 0)),
            pl.BlockSpec((R, 1), lambda r: (r, 0)),
        ],
        out_specs=pl.BlockSpec((R, D), lambda r: (r, 0)),
        out_shape=jax.ShapeDtypeStruct((N, D), jnp.float32),
    )(x, w, dinv)


def _junk_mask(r):
    rows = r * R + lax.broadcasted_iota(jnp.int32, (R, 1), 0)
    return jnp.where(rows < JUNK_ROWS, 1.0, 0.0)


def _layer_body(sp_ref, y_ref, dinv_ref, b_ref, w_ref, out_ref):
    h = sp_ref[0] + sp_ref[1] - _junk_mask(pl.program_id(0)) * y_ref[...]
    h = jnp.maximum(dinv_ref[...] * h + b_ref[...], 0.0)
    out_ref[...] = dinv_ref[...] * jnp.dot(
        h, w_ref[...], preferred_element_type=jnp.float32)


def _tc_layer(s_part, y, dinv, b, w):
    return pl.pallas_call(
        _layer_body,
        grid=(N // R,),
        in_specs=[
            pl.BlockSpec((2, R, D), lambda r: (0, r, 0)),
            pl.BlockSpec((R, D), lambda r: (r, 0)),
            pl.BlockSpec((R, 1), lambda r: (r, 0)),
            pl.BlockSpec((1, D), lambda r: (0, 0)),
            pl.BlockSpec((D, D), lambda r: (0, 0)),
        ],
        out_specs=pl.BlockSpec((R, D), lambda r: (r, 0)),
        out_shape=jax.ShapeDtypeStruct((N, D), jnp.float32),
    )(s_part, y, dinv, b, w)


def _z_body(sp_ref, y_ref, dinv_ref, b_ref, out_ref):
    s = sp_ref[0] + sp_ref[1] - _junk_mask(pl.program_id(0)) * y_ref[...]
    out_ref[...] = dinv_ref[...] * s + b_ref[...]


def _tc_z(s_part, y, dinv, b):
    return pl.pallas_call(
        _z_body,
        grid=(N // R,),
        in_specs=[
            pl.BlockSpec((2, R, D), lambda r: (0, r, 0)),
            pl.BlockSpec((R, D), lambda r: (r, 0)),
            pl.BlockSpec((R, 1), lambda r: (r, 0)),
            pl.BlockSpec((1, D), lambda r: (0, 0)),
        ],
        out_specs=pl.BlockSpec((R, D), lambda r: (r, 0)),
        out_shape=jax.ShapeDtypeStruct((N, D), jnp.float32),
    )(s_part, y, dinv, b)


def _final_body(pp_ref, cnt_ref, wl_ref, bl_ref, out_ref):
    pooled = (pp_ref[0] + pp_ref[1]) / jnp.maximum(cnt_ref[:, :1], 1.0)
    out_ref[...] = jnp.dot(
        pooled, wl_ref[...], preferred_element_type=jnp.float32) + bl_ref[...]


def _tc_final(pool_part, cnt16, wlin, blin):
    return pl.pallas_call(
        _final_body,
        out_shape=jax.ShapeDtypeStruct((G, wlin.shape[1]), jnp.float32),
    )(pool_part, cnt16, wlin, blin)


def kernel(x, edge_index, batch, W1, b1, W2, b2, W3, b3, Wlin, blin):
    dst2d = edge_index[1].reshape(E // 50, 50)
    batch2d = batch.reshape(NPC, PC)
    zeros = jnp.zeros((N, D), jnp.float32)

    sr = edge_index[0].reshape(32, E // 32)
    dr = edge_index[1].reshape(32, E // 32)
    jr = (112 * jnp.arange(32, dtype=jnp.int32)[:, None]
          + jnp.arange(112, dtype=jnp.int32)[None, :])
    zr = jnp.zeros((32, CE), jnp.int32)
    srcp = jnp.concatenate([sr, jr, zr], axis=1)
    dstp = jnp.concatenate([dr, jr, zr], axis=1)
    pk2d = jnp.bitwise_or(srcp, dstp << 16).reshape(32 * 80, CE)

    ones_in = jnp.ones((PC, D), jnp.float32)
    deg2, cnt = _sc_precompute(dst2d, batch2d, ones_in, zeros)
    dinv = _tc_dinv(deg2.reshape(2, N, D))

    y1 = _tc_mm1(x, W1, dinv)
    s1 = _sc_edge(y1, pk2d, zeros).reshape(2, N, D)
    y2 = _tc_layer(s1, y1, dinv, b1.reshape(1, D), W2)
    s2 = _sc_edge(y2, pk2d, zeros).reshape(2, N, D)
    y3 = _tc_layer(s2, y2, dinv, b2.reshape(1, D), W3)
    s3 = _sc_edge(y3, pk2d, zeros).reshape(2, N, D)

    z = _tc_z(s3, y3, dinv, b3.reshape(1, D))
    pool_part = _sc_pool(z, batch2d, zeros).reshape(2, G, D)
    return _tc_final(pool_part, cnt, Wlin, blin.reshape(1, blin.shape[0]))


# speedup vs baseline: 20.5194x; 1.0217x over previous
import functools

import jax
import jax.numpy as jnp
from jax import lax
from jax.experimental import pallas as pl
from jax.experimental.pallas import tpu as pltpu, tpu_sc as plsc

N = 10000
E = 320000
D = 128
G = 64
PC = 80
NPC = N // PC

_MESH = plsc.VectorSubcoreMesh(core_axis_name="c", subcore_axis_name="s")


@functools.partial(
    pl.kernel,
    out_type=(
        jax.ShapeDtypeStruct((2 * N, D), jnp.float32),
        jax.ShapeDtypeStruct((G, D), jnp.float32),
    ),
    mesh=_MESH,
    scratch_types=[
        pltpu.VMEM((200, 50), jnp.int32),
        pltpu.VMEM((8, PC), jnp.int32),
        pltpu.VMEM((PC, D), jnp.float32),
        pltpu.VMEM((PC, D), jnp.float32),
        pltpu.SemaphoreType.DMA,
        pltpu.VMEM_SHARED((N, D), jnp.float32),
        pltpu.VMEM_SHARED((G, D), jnp.float32),
    ],
)
def _sc_precompute(dst2d, batch2d, ones_in, zeros, deg_out, cnt_out,
                   dbuf, bbuf, ones, zbuf, dsem, deg_sh, cnt_sh):
    c = lax.axis_index("c")
    s = lax.axis_index("s")

    pltpu.sync_copy(ones_in, ones)
    pltpu.sync_copy(zeros.at[pl.ds(0, PC)], zbuf)

    for k in range(8):
        ch = s + 16 * k

        @pl.when(ch < NPC)
        def _():
            pltpu.sync_copy(zbuf, deg_sh.at[pl.ds(ch * PC, PC)])

    @pl.when(jnp.logical_and(c == 0, s == 0))
    def _():
        pltpu.sync_copy(zbuf.at[pl.ds(0, G)], cnt_sh)

    base = c * 3200 + s * 200
    pltpu.sync_copy(dst2d.at[pl.ds(base, 200)], dbuf)

    @pl.when(c == 0)
    def _():
        for k in range(8):
            ch = s + 16 * k

            @pl.when(ch < NPC)
            def _():
                pltpu.sync_copy(batch2d.at[ch], bbuf.at[k])

    plsc.subcore_barrier()

    def edge_body(j, _):
        @pl.when(j >= 4)
        def _():
            pltpu.make_async_copy(
                ones.at[pl.ds(0, 50)], deg_sh.at[dbuf.at[j]], dsem).wait()

        pltpu.async_copy(
            ones.at[pl.ds(0, 50)], deg_sh.at[dbuf.at[j]], dsem, add=True)
        return 0
    lax.fori_loop(0, 200, edge_body, 0)
    for _ in range(4):
        pltpu.make_async_copy(
            ones.at[pl.ds(0, 50)], deg_sh.at[dbuf.at[0]], dsem).wait()

    @pl.when(c == 0)
    def _():
        for k in range(8):
            ch = s + 16 * k

            @pl.when(ch < NPC)
            def _():
                pltpu.sync_copy(ones.at[pl.ds(0, PC)],
                                cnt_sh.at[bbuf.at[k]], add=True)

    plsc.subcore_barrier()

    for k in range(8):
        ch = s + 16 * k

        @pl.when(ch < NPC)
        def _():
            sl = pl.ds(ch * PC, PC)
            pltpu.sync_copy(deg_sh.at[sl], zbuf)
            pltpu.sync_copy(zbuf, deg_out.at[pl.ds(c * N + ch * PC, PC)])

    @pl.when(jnp.logical_and(c == 0, s == 0))
    def _():
        pltpu.sync_copy(cnt_sh, zbuf.at[pl.ds(0, G)])
        pltpu.sync_copy(zbuf.at[pl.ds(0, G)], cnt_out)


CE = 128
CPT = 79
JUNK_ROWS = 3584


@functools.partial(
    pl.kernel,
    out_type=jax.ShapeDtypeStruct((2 * N, D), jnp.float32),
    mesh=_MESH,
    scratch_types=[
        pltpu.VMEM((80, CE), jnp.int32),
        pltpu.VMEM((2, CE), jnp.int32),
        pltpu.VMEM((2, CE), jnp.int32),
        pltpu.VMEM((CE, D), jnp.float32),
        pltpu.VMEM((CE, D), jnp.float32),
        pltpu.SemaphoreType.DMA,
        pltpu.SemaphoreType.DMA,
        pltpu.VMEM_SHARED((N, D), jnp.float32),
    ],
)
def _sc_edge(y, pk2d, s_out,
             ipack, sidx, didx, rb0, rb1, gsem, ssem, s_sh):
    c = lax.axis_index("c")
    s = lax.axis_index("s")
    wid = c * 16 + s
    rbs = [rb0, rb1]
    stage = rb0.at[pl.ds(0, PC)]

    for k in range(8):
        ch = s + 16 * k

        @pl.when(ch < NPC)
        def _():
            sl = pl.ds(ch * PC, PC)
            pltpu.sync_copy(y.at[sl], stage)
            pltpu.sync_copy(stage, s_sh.at[sl])

    pltpu.sync_copy(pk2d.at[pl.ds(wid * 80, 80)], ipack)

    plsc.subcore_barrier()

    def unpack(j, slot):
        for v in range(8):
            pk = ipack[j, pl.ds(16 * v, 16)]
            sidx[slot, pl.ds(16 * v, 16)] = jnp.bitwise_and(pk, 0xFFFF)
            didx[slot, pl.ds(16 * v, 16)] = lax.shift_right_logical(pk, 16)

    unpack(0, 0)
    pltpu.async_copy(y.at[sidx.at[0]], rb0, gsem)

    def edge_body(i, _):
        for b in range(2):
            j = 2 * i + b
            rb, ro = rbs[b], rbs[1 - b]
            pltpu.make_async_copy(y.at[sidx.at[b]], rb, gsem).wait()
            pltpu.async_copy(rb, s_sh.at[didx.at[b]], ssem, add=True)

            @pl.when(j >= 1)
            def _():
                pltpu.make_async_copy(ro, s_sh.at[didx.at[1 - b]], ssem).wait()

            unpack(j + 1, 1 - b)
            pltpu.async_copy(y.at[sidx.at[1 - b]], ro, gsem)
        return 0
    lax.fori_loop(0, (CPT - 1) // 2, edge_body, 0)

    pltpu.make_async_copy(y.at[sidx.at[0]], rb0, gsem).wait()
    pltpu.async_copy(rb0, s_sh.at[didx.at[0]], ssem, add=True)
    pltpu.make_async_copy(rb1, s_sh.at[didx.at[1]], ssem).wait()
    pltpu.make_async_copy(rb0, s_sh.at[didx.at[0]], ssem).wait()

    plsc.subcore_barrier()

    for k in range(8):
        ch = s + 16 * k

        @pl.when(ch < NPC)
        def _():
            sl = pl.ds(ch * PC, PC)
            pltpu.sync_copy(s_sh.at[sl], stage)
            pltpu.sync_copy(stage, s_out.at[pl.ds(c * N + ch * PC, PC)])


@functools.partial(
    pl.kernel,
    out_type=jax.ShapeDtypeStruct((2 * G, D), jnp.float32),
    mesh=_MESH,
    scratch_types=[
        pltpu.VMEM((4, PC), jnp.int32),
        pltpu.VMEM((PC, D), jnp.float32),
        pltpu.VMEM_SHARED((G, D), jnp.float32),
    ],
)
def _sc_pool(z, batch2d, zeros, p_out, bbuf, rbuf, p_sh):
    c = lax.axis_index("c")
    s = lax.axis_index("s")
    wid = c * 16 + s

    @pl.when(s == 0)
    def _():
        pltpu.sync_copy(zeros.at[pl.ds(0, G)], rbuf.at[pl.ds(0, G)])
        pltpu.sync_copy(rbuf.at[pl.ds(0, G)], p_sh)

    for k in range(4):
        ch = wid + 32 * k

        @pl.when(ch < NPC)
        def _():
            pltpu.sync_copy(batch2d.at[ch], bbuf.at[k])

    plsc.subcore_barrier()

    for k in range(4):
        ch = wid + 32 * k

        @pl.when(ch < NPC)
        def _():
            pltpu.sync_copy(z.at[pl.ds(ch * PC, PC)], rbuf)
            pltpu.sync_copy(rbuf, p_sh.at[bbuf.at[k]], add=True)

    plsc.subcore_barrier()

    @pl.when(s == 0)
    def _():
        pltpu.sync_copy(p_sh, rbuf.at[pl.ds(0, G)])
        pltpu.sync_copy(rbuf.at[pl.ds(0, G)], p_out.at[pl.ds(c * G, G)])


R = 1000


def _mm1_body(deg_ref, x_ref, w_ref, y_ref, dinv_ref):
    deg = deg_ref[0][:, :1] + deg_ref[1][:, :1] + 1.0
    dinv = lax.rsqrt(deg)
    dinv_ref[...] = dinv
    y_ref[...] = dinv * jnp.dot(
        x_ref[...], w_ref[...], preferred_element_type=jnp.float32)


def _tc_mm1(deg2, x, w):
    return pl.pallas_call(
        _mm1_body,
        grid=(N // R,),
        in_specs=[
            pl.BlockSpec((2, R, D), lambda r: (0, r, 0)),
            pl.BlockSpec((R, D), lambda r: (r, 0)),
            pl.BlockSpec((D, D), lambda r: (0, 0)),
        ],
        out_specs=[
            pl.BlockSpec((R, D), lambda r: (r, 0)),
            pl.BlockSpec((R, 1), lambda r: (r, 0)),
        ],
        out_shape=[
            jax.ShapeDtypeStruct((N, D), jnp.float32),
            jax.ShapeDtypeStruct((N, 1), jnp.float32),
        ],
    )(deg2, x, w)


def _junk_mask(r):
    rows = r * R + lax.broadcasted_iota(jnp.int32, (R, 1), 0)
    return jnp.where(rows < JUNK_ROWS, 2.0, 1.0)


def _layer_body(sp_ref, y_ref, dinv_ref, b_ref, w_ref, out_ref):
    h = sp_ref[0] + sp_ref[1] - _junk_mask(pl.program_id(0)) * y_ref[...]
    h = jnp.maximum(dinv_ref[...] * h + b_ref[...], 0.0)
    out_ref[...] = dinv_ref[...] * jnp.dot(
        h, w_ref[...], preferred_element_type=jnp.float32)


def _tc_layer(s_part, y, dinv, b, w):
    return pl.pallas_call(
        _layer_body,
        grid=(N // R,),
        in_specs=[
            pl.BlockSpec((2, R, D), lambda r: (0, r, 0)),
            pl.BlockSpec((R, D), lambda r: (r, 0)),
            pl.BlockSpec((R, 1), lambda r: (r, 0)),
            pl.BlockSpec((1, D), lambda r: (0, 0)),
            pl.BlockSpec((D, D), lambda r: (0, 0)),
        ],
        out_specs=pl.BlockSpec((R, D), lambda r: (r, 0)),
        out_shape=jax.ShapeDtypeStruct((N, D), jnp.float32),
    )(s_part, y, dinv, b, w)


def _z_body(sp_ref, y_ref, dinv_ref, b_ref, out_ref):
    s = sp_ref[0] + sp_ref[1] - _junk_mask(pl.program_id(0)) * y_ref[...]
    out_ref[...] = dinv_ref[...] * s + b_ref[...]


def _tc_z(s_part, y, dinv, b):
    return pl.pallas_call(
        _z_body,
        grid=(N // R,),
        in_specs=[
            pl.BlockSpec((2, R, D), lambda r: (0, r, 0)),
            pl.BlockSpec((R, D), lambda r: (r, 0)),
            pl.BlockSpec((R, 1), lambda r: (r, 0)),
            pl.BlockSpec((1, D), lambda r: (0, 0)),
        ],
        out_specs=pl.BlockSpec((R, D), lambda r: (r, 0)),
        out_shape=jax.ShapeDtypeStruct((N, D), jnp.float32),
    )(s_part, y, dinv, b)


def _final_body(pp_ref, cnt_ref, wl_ref, bl_ref, out_ref):
    pooled = (pp_ref[0] + pp_ref[1]) / jnp.maximum(cnt_ref[:, :1], 1.0)
    out_ref[...] = jnp.dot(
        pooled, wl_ref[...], preferred_element_type=jnp.float32) + bl_ref[...]


def _tc_final(pool_part, cnt16, wlin, blin):
    return pl.pallas_call(
        _final_body,
        out_shape=jax.ShapeDtypeStruct((G, wlin.shape[1]), jnp.float32),
    )(pool_part, cnt16, wlin, blin)


def kernel(x, edge_index, batch, W1, b1, W2, b2, W3, b3, Wlin, blin):
    dst2d = edge_index[1].reshape(E // 50, 50)
    batch2d = batch.reshape(NPC, PC)
    zeros = jnp.zeros((N, D), jnp.float32)

    sr = edge_index[0].reshape(32, E // 32)
    dr = edge_index[1].reshape(32, E // 32)
    jr = (112 * jnp.arange(32, dtype=jnp.int32)[:, None]
          + jnp.arange(112, dtype=jnp.int32)[None, :])
    zr = jnp.zeros((32, CE), jnp.int32)
    srcp = jnp.concatenate([sr, jr, zr], axis=1)
    dstp = jnp.concatenate([dr, jr, zr], axis=1)
    pk2d = jnp.bitwise_or(srcp, dstp << 16).reshape(32 * 80, CE)

    ones_in = jnp.ones((PC, D), jnp.float32)
    deg2, cnt = _sc_precompute(dst2d, batch2d, ones_in, zeros)

    y1, dinv = _tc_mm1(deg2.reshape(2, N, D), x, W1)
    s1 = _sc_edge(y1, pk2d).reshape(2, N, D)
    y2 = _tc_layer(s1, y1, dinv, b1.reshape(1, D), W2)
    s2 = _sc_edge(y2, pk2d).reshape(2, N, D)
    y3 = _tc_layer(s2, y2, dinv, b2.reshape(1, D), W3)
    s3 = _sc_edge(y3, pk2d).reshape(2, N, D)

    z = _tc_z(s3, y3, dinv, b3.reshape(1, D))
    pool_part = _sc_pool(z, batch2d, zeros).reshape(2, G, D)
    return _tc_final(pool_part, cnt, Wlin, blin.reshape(1, blin.shape[0]))


# speedup vs baseline: 23.0590x; 1.1238x over previous
import functools

import jax
import jax.numpy as jnp
from jax import lax
from jax.experimental import pallas as pl
from jax.experimental.pallas import tpu as pltpu, tpu_sc as plsc

N = 10000
E = 320000
D = 128
G = 64
PC = 80
NPC = N // PC

_MESH = plsc.VectorSubcoreMesh(core_axis_name="c", subcore_axis_name="s")


@functools.partial(
    pl.kernel,
    out_type=(
        jax.ShapeDtypeStruct((2 * N, D), jnp.float32),
        jax.ShapeDtypeStruct((G, D), jnp.float32),
    ),
    mesh=_MESH,
    scratch_types=[
        pltpu.VMEM((200, 50), jnp.int32),
        pltpu.VMEM((8, PC), jnp.int32),
        pltpu.VMEM((PC, D), jnp.float32),
        pltpu.VMEM((PC, D), jnp.float32),
        pltpu.SemaphoreType.DMA,
        pltpu.VMEM_SHARED((N, D), jnp.float32),
        pltpu.VMEM_SHARED((G, D), jnp.float32),
    ],
)
def _sc_precompute(dst2d, batch2d, ones_in, zeros, deg_out, cnt_out,
                   dbuf, bbuf, ones, zbuf, dsem, deg_sh, cnt_sh):
    c = lax.axis_index("c")
    s = lax.axis_index("s")

    pltpu.sync_copy(ones_in, ones)
    pltpu.sync_copy(zeros.at[pl.ds(0, PC)], zbuf)

    for k in range(8):
        ch = s + 16 * k

        @pl.when(ch < NPC)
        def _():
            pltpu.sync_copy(zbuf, deg_sh.at[pl.ds(ch * PC, PC)])

    @pl.when(jnp.logical_and(c == 0, s == 0))
    def _():
        pltpu.sync_copy(zbuf.at[pl.ds(0, G)], cnt_sh)

    base = c * 3200 + s * 200
    pltpu.sync_copy(dst2d.at[pl.ds(base, 200)], dbuf)

    @pl.when(c == 0)
    def _():
        for k in range(8):
            ch = s + 16 * k

            @pl.when(ch < NPC)
            def _():
                pltpu.sync_copy(batch2d.at[ch], bbuf.at[k])

    plsc.subcore_barrier()

    def edge_body(j, _):
        @pl.when(j >= 4)
        def _():
            pltpu.make_async_copy(
                ones.at[pl.ds(0, 50)], deg_sh.at[dbuf.at[j]], dsem).wait()

        pltpu.async_copy(
            ones.at[pl.ds(0, 50)], deg_sh.at[dbuf.at[j]], dsem, add=True)
        return 0
    lax.fori_loop(0, 200, edge_body, 0)
    for _ in range(4):
        pltpu.make_async_copy(
            ones.at[pl.ds(0, 50)], deg_sh.at[dbuf.at[0]], dsem).wait()

    @pl.when(c == 0)
    def _():
        for k in range(8):
            ch = s + 16 * k

            @pl.when(ch < NPC)
            def _():
                pltpu.sync_copy(ones.at[pl.ds(0, PC)],
                                cnt_sh.at[bbuf.at[k]], add=True)

    plsc.subcore_barrier()

    for k in range(8):
        ch = s + 16 * k

        @pl.when(ch < NPC)
        def _():
            sl = pl.ds(ch * PC, PC)
            pltpu.sync_copy(deg_sh.at[sl], zbuf)
            pltpu.sync_copy(zbuf, deg_out.at[pl.ds(c * N + ch * PC, PC)])

    @pl.when(jnp.logical_and(c == 0, s == 0))
    def _():
        pltpu.sync_copy(cnt_sh, zbuf.at[pl.ds(0, G)])
        pltpu.sync_copy(zbuf.at[pl.ds(0, G)], cnt_out)


CE = 128
CPT = 79
JUNK_ROWS = 3584


@functools.partial(
    pl.kernel,
    out_type=jax.ShapeDtypeStruct((2 * N, D), jnp.float32),
    mesh=_MESH,
    scratch_types=[
        pltpu.VMEM((80, CE), jnp.int32),
        pltpu.VMEM((4, CE // 2), jnp.int32),
        pltpu.VMEM((4, CE // 2), jnp.int32),
        pltpu.VMEM((CE, D), jnp.float32),
        pltpu.VMEM((CE, D), jnp.float32),
        pltpu.SemaphoreType.DMA,
        pltpu.SemaphoreType.DMA,
        pltpu.VMEM_SHARED((N, D), jnp.float32),
    ],
)
def _sc_edge(y, pk2d, s_out,
             ipack, sidx, didx, rb0, rb1, gsem, ssem, s_sh):
    c = lax.axis_index("c")
    s = lax.axis_index("s")
    wid = c * 16 + s
    rbs = [rb0, rb1]
    stage = rb0.at[pl.ds(0, PC)]

    for k in range(8):
        ch = s + 16 * k

        @pl.when(ch < NPC)
        def _():
            sl = pl.ds(ch * PC, PC)
            pltpu.sync_copy(y.at[sl], stage)
            pltpu.sync_copy(stage, s_sh.at[sl])

    pltpu.sync_copy(pk2d.at[pl.ds(wid * 80, 80)], ipack)

    plsc.subcore_barrier()

    HC = CE // 2
    NH = 2 * CPT
    slots = [rb0.at[pl.ds(0, HC)], rb0.at[pl.ds(HC, HC)],
             rb1.at[pl.ds(0, HC)], rb1.at[pl.ds(HC, HC)]]

    def unpack(j, b):
        row = j // 2
        cb = HC * (b % 2)
        for v in range(4):
            pk = ipack[row, pl.ds(cb + 16 * v, 16)]
            sidx[b, pl.ds(16 * v, 16)] = jnp.bitwise_and(pk, 0xFFFF)
            didx[b, pl.ds(16 * v, 16)] = lax.shift_right_logical(pk, 16)

    def gather(j, b):
        pltpu.async_copy(y.at[sidx.at[b]], slots[b], gsem)

    def wait_gather(b):
        pltpu.make_async_copy(y.at[sidx.at[b]], slots[b], gsem).wait()

    def scatter(b):
        pltpu.async_copy(slots[b], s_sh.at[didx.at[b]], ssem, add=True)

    def wait_scatter(b):
        pltpu.make_async_copy(slots[b], s_sh.at[didx.at[b]], ssem).wait()

    unpack(0, 0)
    gather(0, 0)
    unpack(1, 1)
    gather(1, 1)

    def edge_body(i, _):
        for b in range(4):
            j = 4 * i + b
            wait_gather(b)
            scatter(b)

            @pl.when(j >= 2)
            def _():
                wait_scatter((b + 2) % 4)

            @pl.when(j + 2 < NH)
            def _():
                unpack(j + 2, (b + 2) % 4)
                gather(j + 2, (b + 2) % 4)
        return 0
    lax.fori_loop(0, NH // 4, edge_body, 0)

    for b in (0, 1):
        wait_gather(b)
        scatter(b)
        wait_scatter((b + 2) % 4)
    wait_scatter(0)
    wait_scatter(1)

    plsc.subcore_barrier()

    for k in range(8):
        ch = s + 16 * k

        @pl.when(ch < NPC)
        def _():
            sl = pl.ds(ch * PC, PC)
            pltpu.sync_copy(s_sh.at[sl], stage)
            pltpu.sync_copy(stage, s_out.at[pl.ds(c * N + ch * PC, PC)])


@functools.partial(
    pl.kernel,
    out_type=jax.ShapeDtypeStruct((2 * G, D), jnp.float32),
    mesh=_MESH,
    scratch_types=[
        pltpu.VMEM((4, PC), jnp.int32),
        pltpu.VMEM((PC, D), jnp.float32),
        pltpu.VMEM_SHARED((G, D), jnp.float32),
    ],
)
def _sc_pool(z, batch2d, zeros, p_out, bbuf, rbuf, p_sh):
    c = lax.axis_index("c")
    s = lax.axis_index("s")
    wid = c * 16 + s

    @pl.when(s == 0)
    def _():
        pltpu.sync_copy(zeros.at[pl.ds(0, G)], rbuf.at[pl.ds(0, G)])
        pltpu.sync_copy(rbuf.at[pl.ds(0, G)], p_sh)

    for k in range(4):
        ch = wid + 32 * k

        @pl.when(ch < NPC)
        def _():
            pltpu.sync_copy(batch2d.at[ch], bbuf.at[k])

    plsc.subcore_barrier()

    for k in range(4):
        ch = wid + 32 * k

        @pl.when(ch < NPC)
        def _():
            pltpu.sync_copy(z.at[pl.ds(ch * PC, PC)], rbuf)
            pltpu.sync_copy(rbuf, p_sh.at[bbuf.at[k]], add=True)

    plsc.subcore_barrier()

    @pl.when(s == 0)
    def _():
        pltpu.sync_copy(p_sh, rbuf.at[pl.ds(0, G)])
        pltpu.sync_copy(rbuf.at[pl.ds(0, G)], p_out.at[pl.ds(c * G, G)])


R = 1000


def _mm1_body(deg_ref, x_ref, w_ref, y_ref, dinv_ref):
    deg = deg_ref[0][:, :1] + deg_ref[1][:, :1] + 1.0
    dinv = lax.rsqrt(deg)
    dinv_ref[...] = dinv
    y_ref[...] = dinv * jnp.dot(
        x_ref[...], w_ref[...], preferred_element_type=jnp.float32)


def _tc_mm1(deg2, x, w):
    return pl.pallas_call(
        _mm1_body,
        grid=(N // R,),
        in_specs=[
            pl.BlockSpec((2, R, D), lambda r: (0, r, 0)),
            pl.BlockSpec((R, D), lambda r: (r, 0)),
            pl.BlockSpec((D, D), lambda r: (0, 0)),
        ],
        out_specs=[
            pl.BlockSpec((R, D), lambda r: (r, 0)),
            pl.BlockSpec((R, 1), lambda r: (r, 0)),
        ],
        out_shape=[
            jax.ShapeDtypeStruct((N, D), jnp.float32),
            jax.ShapeDtypeStruct((N, 1), jnp.float32),
        ],
    )(deg2, x, w)


def _junk_mask(r):
    rows = r * R + lax.broadcasted_iota(jnp.int32, (R, 1), 0)
    return jnp.where(rows < JUNK_ROWS, 2.0, 1.0)


def _layer_body(sp_ref, y_ref, dinv_ref, b_ref, w_ref, out_ref):
    h = sp_ref[0] + sp_ref[1] - _junk_mask(pl.program_id(0)) * y_ref[...]
    h = jnp.maximum(dinv_ref[...] * h + b_ref[...], 0.0)
    out_ref[...] = dinv_ref[...] * jnp.dot(
        h, w_ref[...], preferred_element_type=jnp.float32)


def _tc_layer(s_part, y, dinv, b, w):
    return pl.pallas_call(
        _layer_body,
        grid=(N // R,),
        in_specs=[
            pl.BlockSpec((2, R, D), lambda r: (0, r, 0)),
            pl.BlockSpec((R, D), lambda r: (r, 0)),
            pl.BlockSpec((R, 1), lambda r: (r, 0)),
            pl.BlockSpec((1, D), lambda r: (0, 0)),
            pl.BlockSpec((D, D), lambda r: (0, 0)),
        ],
        out_specs=pl.BlockSpec((R, D), lambda r: (r, 0)),
        out_shape=jax.ShapeDtypeStruct((N, D), jnp.float32),
    )(s_part, y, dinv, b, w)


def _z_body(sp_ref, y_ref, dinv_ref, b_ref, out_ref):
    s = sp_ref[0] + sp_ref[1] - _junk_mask(pl.program_id(0)) * y_ref[...]
    out_ref[...] = dinv_ref[...] * s + b_ref[...]


def _tc_z(s_part, y, dinv, b):
    return pl.pallas_call(
        _z_body,
        grid=(N // R,),
        in_specs=[
            pl.BlockSpec((2, R, D), lambda r: (0, r, 0)),
            pl.BlockSpec((R, D), lambda r: (r, 0)),
            pl.BlockSpec((R, 1), lambda r: (r, 0)),
            pl.BlockSpec((1, D), lambda r: (0, 0)),
        ],
        out_specs=pl.BlockSpec((R, D), lambda r: (r, 0)),
        out_shape=jax.ShapeDtypeStruct((N, D), jnp.float32),
    )(s_part, y, dinv, b)


def _final_body(pp_ref, cnt_ref, wl_ref, bl_ref, out_ref):
    pooled = (pp_ref[0] + pp_ref[1]) / jnp.maximum(cnt_ref[:, :1], 1.0)
    out_ref[...] = jnp.dot(
        pooled, wl_ref[...], preferred_element_type=jnp.float32) + bl_ref[...]


def _tc_final(pool_part, cnt16, wlin, blin):
    return pl.pallas_call(
        _final_body,
        out_shape=jax.ShapeDtypeStruct((G, wlin.shape[1]), jnp.float32),
    )(pool_part, cnt16, wlin, blin)


def kernel(x, edge_index, batch, W1, b1, W2, b2, W3, b3, Wlin, blin):
    dst2d = edge_index[1].reshape(E // 50, 50)
    batch2d = batch.reshape(NPC, PC)
    zeros = jnp.zeros((N, D), jnp.float32)

    sr = edge_index[0].reshape(32, E // 32)
    dr = edge_index[1].reshape(32, E // 32)
    jr = (112 * jnp.arange(32, dtype=jnp.int32)[:, None]
          + jnp.arange(112, dtype=jnp.int32)[None, :])
    zr = jnp.zeros((32, CE), jnp.int32)
    srcp = jnp.concatenate([sr, jr, zr], axis=1)
    dstp = jnp.concatenate([dr, jr, zr], axis=1)
    pk2d = jnp.bitwise_or(srcp, dstp << 16).reshape(32 * 80, CE)

    ones_in = jnp.ones((PC, D), jnp.float32)
    deg2, cnt = _sc_precompute(dst2d, batch2d, ones_in, zeros)

    y1, dinv = _tc_mm1(deg2.reshape(2, N, D), x, W1)
    s1 = _sc_edge(y1, pk2d).reshape(2, N, D)
    y2 = _tc_layer(s1, y1, dinv, b1.reshape(1, D), W2)
    s2 = _sc_edge(y2, pk2d).reshape(2, N, D)
    y3 = _tc_layer(s2, y2, dinv, b2.reshape(1, D), W3)
    s3 = _sc_edge(y3, pk2d).reshape(2, N, D)

    z = _tc_z(s3, y3, dinv, b3.reshape(1, D))
    pool_part = _sc_pool(z, batch2d, zeros).reshape(2, G, D)
    return _tc_final(pool_part, cnt, Wlin, blin.reshape(1, blin.shape[0]))


# speedup vs baseline: 24.4048x; 1.0584x over previous
import functools

import jax
import jax.numpy as jnp
from jax import lax
from jax.experimental import pallas as pl
from jax.experimental.pallas import tpu as pltpu, tpu_sc as plsc

N = 10000
E = 320000
D = 128
G = 64
PC = 80
NPC = N // PC

_MESH = plsc.VectorSubcoreMesh(core_axis_name="c", subcore_axis_name="s")


@functools.partial(
    pl.kernel,
    out_type=(
        jax.ShapeDtypeStruct((2 * N, D), jnp.float32),
        jax.ShapeDtypeStruct((G, D), jnp.float32),
    ),
    mesh=_MESH,
    scratch_types=[
        pltpu.VMEM((200, 50), jnp.int32),
        pltpu.VMEM((8, PC), jnp.int32),
        pltpu.VMEM((PC, D), jnp.float32),
        pltpu.VMEM((PC, D), jnp.float32),
        pltpu.SemaphoreType.DMA,
        pltpu.VMEM_SHARED((N, D), jnp.float32),
        pltpu.VMEM_SHARED((G, D), jnp.float32),
    ],
)
def _sc_precompute(dst2d, batch2d, ones_in, zeros, deg_out, cnt_out,
                   dbuf, bbuf, ones, zbuf, dsem, deg_sh, cnt_sh):
    c = lax.axis_index("c")
    s = lax.axis_index("s")

    pltpu.sync_copy(ones_in, ones)
    pltpu.sync_copy(zeros.at[pl.ds(0, PC)], zbuf)

    for k in range(8):
        ch = s + 16 * k

        @pl.when(ch < NPC)
        def _():
            pltpu.sync_copy(zbuf, deg_sh.at[pl.ds(ch * PC, PC)])

    @pl.when(jnp.logical_and(c == 0, s == 0))
    def _():
        pltpu.sync_copy(zbuf.at[pl.ds(0, G)], cnt_sh)

    base = c * 3200 + s * 200
    pltpu.sync_copy(dst2d.at[pl.ds(base, 200)], dbuf)

    @pl.when(c == 0)
    def _():
        for k in range(8):
            ch = s + 16 * k

            @pl.when(ch < NPC)
            def _():
                pltpu.sync_copy(batch2d.at[ch], bbuf.at[k])

    plsc.subcore_barrier()

    def edge_body(j, _):
        @pl.when(j >= 4)
        def _():
            pltpu.make_async_copy(
                ones.at[pl.ds(0, 50)], deg_sh.at[dbuf.at[j]], dsem).wait()

        pltpu.async_copy(
            ones.at[pl.ds(0, 50)], deg_sh.at[dbuf.at[j]], dsem, add=True)
        return 0
    lax.fori_loop(0, 200, edge_body, 0)
    for _ in range(4):
        pltpu.make_async_copy(
            ones.at[pl.ds(0, 50)], deg_sh.at[dbuf.at[0]], dsem).wait()

    @pl.when(c == 0)
    def _():
        for k in range(8):
            ch = s + 16 * k

            @pl.when(ch < NPC)
            def _():
                pltpu.sync_copy(ones.at[pl.ds(0, PC)],
                                cnt_sh.at[bbuf.at[k]], add=True)

    plsc.subcore_barrier()

    for k in range(8):
        ch = s + 16 * k

        @pl.when(ch < NPC)
        def _():
            sl = pl.ds(ch * PC, PC)
            pltpu.sync_copy(deg_sh.at[sl], zbuf)
            pltpu.sync_copy(zbuf, deg_out.at[pl.ds(c * N + ch * PC, PC)])

    @pl.when(jnp.logical_and(c == 0, s == 0))
    def _():
        pltpu.sync_copy(cnt_sh, zbuf.at[pl.ds(0, G)])
        pltpu.sync_copy(zbuf.at[pl.ds(0, G)], cnt_out)


CE = 128
CPT = 79
JUNK_ROWS = 3584


@functools.partial(
    pl.kernel,
    out_type=jax.ShapeDtypeStruct((2 * N, D), jnp.float32),
    mesh=_MESH,
    scratch_types=[
        pltpu.VMEM((80, CE), jnp.int32),
        pltpu.VMEM((8, CE // 4), jnp.int32),
        pltpu.VMEM((8, CE // 4), jnp.int32),
        pltpu.VMEM((CE, D), jnp.float32),
        pltpu.VMEM((CE, D), jnp.float32),
        pltpu.SemaphoreType.DMA,
        pltpu.SemaphoreType.DMA,
        pltpu.VMEM_SHARED((N, D), jnp.float32),
    ],
)
def _sc_edge(y, pk2d, s_out,
             ipack, sidx, didx, rb0, rb1, gsem, ssem, s_sh):
    c = lax.axis_index("c")
    s = lax.axis_index("s")
    wid = c * 16 + s
    rbs = [rb0, rb1]
    stage = rb0.at[pl.ds(0, PC)]

    for k in range(8):
        ch = s + 16 * k

        @pl.when(ch < NPC)
        def _():
            sl = pl.ds(ch * PC, PC)
            pltpu.sync_copy(y.at[sl], stage)
            pltpu.sync_copy(stage, s_sh.at[sl])

    pltpu.sync_copy(pk2d.at[pl.ds(wid * 80, 80)], ipack)

    plsc.subcore_barrier()

    HC = CE // 4
    NH = 4 * CPT
    slots = [rb0.at[pl.ds(0, HC)], rb0.at[pl.ds(HC, HC)],
             rb0.at[pl.ds(2 * HC, HC)], rb0.at[pl.ds(3 * HC, HC)],
             rb1.at[pl.ds(0, HC)], rb1.at[pl.ds(HC, HC)],
             rb1.at[pl.ds(2 * HC, HC)], rb1.at[pl.ds(3 * HC, HC)]]

    def unpack(j, b):
        row = j // 4
        cb = HC * (b % 4)
        for v in range(2):
            pk = ipack[row, pl.ds(cb + 16 * v, 16)]
            sidx[b, pl.ds(16 * v, 16)] = jnp.bitwise_and(pk, 0xFFFF)
            didx[b, pl.ds(16 * v, 16)] = lax.shift_right_logical(pk, 16)

    def gather(b):
        pltpu.async_copy(y.at[sidx.at[b]], slots[b], gsem)

    def wait_gather(b):
        pltpu.make_async_copy(y.at[sidx.at[b]], slots[b], gsem).wait()

    def scatter(b):
        pltpu.async_copy(slots[b], s_sh.at[didx.at[b]], ssem, add=True)

    def wait_scatter(b):
        pltpu.make_async_copy(slots[b], s_sh.at[didx.at[b]], ssem).wait()

    for b in range(4):
        unpack(b, b)
        gather(b)

    def edge_body(i, _):
        for b in range(8):
            j = 8 * i + b
            wait_gather(b)
            scatter(b)

            @pl.when(j >= 4)
            def _():
                wait_scatter((b + 4) % 8)

            @pl.when(j + 4 < NH)
            def _():
                unpack(j + 4, (b + 4) % 8)
                gather((b + 4) % 8)
        return 0
    lax.fori_loop(0, NH // 8, edge_body, 0)

    for b in range(4):
        wait_gather(b)
        scatter(b)
        wait_scatter((b + 4) % 8)
    for b in range(4):
        wait_scatter(b)

    plsc.subcore_barrier()

    for k in range(8):
        ch = s + 16 * k

        @pl.when(ch < NPC)
        def _():
            sl = pl.ds(ch * PC, PC)
            pltpu.sync_copy(s_sh.at[sl], stage)
            pltpu.sync_copy(stage, s_out.at[pl.ds(c * N + ch * PC, PC)])


@functools.partial(
    pl.kernel,
    out_type=jax.ShapeDtypeStruct((2 * G, D), jnp.float32),
    mesh=_MESH,
    scratch_types=[
        pltpu.VMEM((4, PC), jnp.int32),
        pltpu.VMEM((PC, D), jnp.float32),
        pltpu.VMEM_SHARED((G, D), jnp.float32),
    ],
)
def _sc_pool(z, batch2d, zeros, p_out, bbuf, rbuf, p_sh):
    c = lax.axis_index("c")
    s = lax.axis_index("s")
    wid = c * 16 + s

    @pl.when(s == 0)
    def _():
        pltpu.sync_copy(zeros.at[pl.ds(0, G)], rbuf.at[pl.ds(0, G)])
        pltpu.sync_copy(rbuf.at[pl.ds(0, G)], p_sh)

    for k in range(4):
        ch = wid + 32 * k

        @pl.when(ch < NPC)
        def _():
            pltpu.sync_copy(batch2d.at[ch], bbuf.at[k])

    plsc.subcore_barrier()

    for k in range(4):
        ch = wid + 32 * k

        @pl.when(ch < NPC)
        def _():
            pltpu.sync_copy(z.at[pl.ds(ch * PC, PC)], rbuf)
            pltpu.sync_copy(rbuf, p_sh.at[bbuf.at[k]], add=True)

    plsc.subcore_barrier()

    @pl.when(s == 0)
    def _():
        pltpu.sync_copy(p_sh, rbuf.at[pl.ds(0, G)])
        pltpu.sync_copy(rbuf.at[pl.ds(0, G)], p_out.at[pl.ds(c * G, G)])


R = 1000


def _mm1_body(deg_ref, x_ref, w_ref, y_ref, dinv_ref):
    deg = deg_ref[0][:, :1] + deg_ref[1][:, :1] + 1.0
    dinv = lax.rsqrt(deg)
    dinv_ref[...] = dinv
    y_ref[...] = dinv * jnp.dot(
        x_ref[...], w_ref[...], preferred_element_type=jnp.float32)


def _tc_mm1(deg2, x, w):
    return pl.pallas_call(
        _mm1_body,
        grid=(N // R,),
        in_specs=[
            pl.BlockSpec((2, R, D), lambda r: (0, r, 0)),
            pl.BlockSpec((R, D), lambda r: (r, 0)),
            pl.BlockSpec((D, D), lambda r: (0, 0)),
        ],
        out_specs=[
            pl.BlockSpec((R, D), lambda r: (r, 0)),
            pl.BlockSpec((R, 1), lambda r: (r, 0)),
        ],
        out_shape=[
            jax.ShapeDtypeStruct((N, D), jnp.float32),
            jax.ShapeDtypeStruct((N, 1), jnp.float32),
        ],
    )(deg2, x, w)


def _junk_mask(r):
    rows = r * R + lax.broadcasted_iota(jnp.int32, (R, 1), 0)
    return jnp.where(rows < JUNK_ROWS, 2.0, 1.0)


def _layer_body(sp_ref, y_ref, dinv_ref, b_ref, w_ref, out_ref):
    h = sp_ref[0] + sp_ref[1] - _junk_mask(pl.program_id(0)) * y_ref[...]
    h = jnp.maximum(dinv_ref[...] * h + b_ref[...], 0.0)
    out_ref[...] = dinv_ref[...] * jnp.dot(
        h, w_ref[...], preferred_element_type=jnp.float32)


def _tc_layer(s_part, y, dinv, b, w):
    return pl.pallas_call(
        _layer_body,
        grid=(N // R,),
        in_specs=[
            pl.BlockSpec((2, R, D), lambda r: (0, r, 0)),
            pl.BlockSpec((R, D), lambda r: (r, 0)),
            pl.BlockSpec((R, 1), lambda r: (r, 0)),
            pl.BlockSpec((1, D), lambda r: (0, 0)),
            pl.BlockSpec((D, D), lambda r: (0, 0)),
        ],
        out_specs=pl.BlockSpec((R, D), lambda r: (r, 0)),
        out_shape=jax.ShapeDtypeStruct((N, D), jnp.float32),
    )(s_part, y, dinv, b, w)


def _z_body(sp_ref, y_ref, dinv_ref, b_ref, out_ref):
    s = sp_ref[0] + sp_ref[1] - _junk_mask(pl.program_id(0)) * y_ref[...]
    out_ref[...] = dinv_ref[...] * s + b_ref[...]


def _tc_z(s_part, y, dinv, b):
    return pl.pallas_call(
        _z_body,
        grid=(N // R,),
        in_specs=[
            pl.BlockSpec((2, R, D), lambda r: (0, r, 0)),
            pl.BlockSpec((R, D), lambda r: (r, 0)),
            pl.BlockSpec((R, 1), lambda r: (r, 0)),
            pl.BlockSpec((1, D), lambda r: (0, 0)),
        ],
        out_specs=pl.BlockSpec((R, D), lambda r: (r, 0)),
        out_shape=jax.ShapeDtypeStruct((N, D), jnp.float32),
    )(s_part, y, dinv, b)


def _final_body(pp_ref, cnt_ref, wl_ref, bl_ref, out_ref):
    pooled = (pp_ref[0] + pp_ref[1]) / jnp.maximum(cnt_ref[:, :1], 1.0)
    out_ref[...] = jnp.dot(
        pooled, wl_ref[...], preferred_element_type=jnp.float32) + bl_ref[...]


def _tc_final(pool_part, cnt16, wlin, blin):
    return pl.pallas_call(
        _final_body,
        out_shape=jax.ShapeDtypeStruct((G, wlin.shape[1]), jnp.float32),
    )(pool_part, cnt16, wlin, blin)


def kernel(x, edge_index, batch, W1, b1, W2, b2, W3, b3, Wlin, blin):
    dst2d = edge_index[1].reshape(E // 50, 50)
    batch2d = batch.reshape(NPC, PC)
    zeros = jnp.zeros((N, D), jnp.float32)

    sr = edge_index[0].reshape(32, E // 32)
    dr = edge_index[1].reshape(32, E // 32)
    jr = (112 * jnp.arange(32, dtype=jnp.int32)[:, None]
          + jnp.arange(112, dtype=jnp.int32)[None, :])
    zr = jnp.zeros((32, CE), jnp.int32)
    srcp = jnp.concatenate([sr, jr, zr], axis=1)
    dstp = jnp.concatenate([dr, jr, zr], axis=1)
    pk2d = jnp.bitwise_or(srcp, dstp << 16).reshape(32 * 80, CE)

    ones_in = jnp.ones((PC, D), jnp.float32)
    deg2, cnt = _sc_precompute(dst2d, batch2d, ones_in, zeros)

    y1, dinv = _tc_mm1(deg2.reshape(2, N, D), x, W1)
    s1 = _sc_edge(y1, pk2d).reshape(2, N, D)
    y2 = _tc_layer(s1, y1, dinv, b1.reshape(1, D), W2)
    s2 = _sc_edge(y2, pk2d).reshape(2, N, D)
    y3 = _tc_layer(s2, y2, dinv, b2.reshape(1, D), W3)
    s3 = _sc_edge(y3, pk2d).reshape(2, N, D)

    z = _tc_z(s3, y3, dinv, b3.reshape(1, D))
    pool_part = _sc_pool(z, batch2d, zeros).reshape(2, G, D)
    return _tc_final(pool_part, cnt, Wlin, blin.reshape(1, blin.shape[0]))


# speedup vs baseline: 24.4244x; 1.0008x over previous
import functools

import jax
import jax.numpy as jnp
from jax import lax
from jax.experimental import pallas as pl
from jax.experimental.pallas import tpu as pltpu, tpu_sc as plsc

N = 10000
E = 320000
D = 128
G = 64
PC = 80
NPC = N // PC

_MESH = plsc.VectorSubcoreMesh(core_axis_name="c", subcore_axis_name="s")


@functools.partial(
    pl.kernel,
    out_type=(
        jax.ShapeDtypeStruct((2 * N, D), jnp.float32),
        jax.ShapeDtypeStruct((G, D), jnp.float32),
    ),
    mesh=_MESH,
    scratch_types=[
        pltpu.VMEM((200, 50), jnp.int32),
        pltpu.VMEM((8, PC), jnp.int32),
        pltpu.VMEM((PC, D), jnp.float32),
        pltpu.VMEM((PC, D), jnp.float32),
        pltpu.SemaphoreType.DMA,
        pltpu.VMEM_SHARED((N, D), jnp.float32),
        pltpu.VMEM_SHARED((G, D), jnp.float32),
    ],
)
def _sc_precompute(dst2d, batch2d, ones_in, zeros, deg_out, cnt_out,
                   dbuf, bbuf, ones, zbuf, dsem, deg_sh, cnt_sh):
    c = lax.axis_index("c")
    s = lax.axis_index("s")

    pltpu.sync_copy(ones_in, ones)
    pltpu.sync_copy(zeros.at[pl.ds(0, PC)], zbuf)

    for k in range(8):
        ch = s + 16 * k

        @pl.when(ch < NPC)
        def _():
            pltpu.sync_copy(zbuf, deg_sh.at[pl.ds(ch * PC, PC)])

    @pl.when(jnp.logical_and(c == 0, s == 0))
    def _():
        pltpu.sync_copy(zbuf.at[pl.ds(0, G)], cnt_sh)

    base = c * 3200 + s * 200
    pltpu.sync_copy(dst2d.at[pl.ds(base, 200)], dbuf)

    @pl.when(c == 0)
    def _():
        for k in range(8):
            ch = s + 16 * k

            @pl.when(ch < NPC)
            def _():
                pltpu.sync_copy(batch2d.at[ch], bbuf.at[k])

    plsc.subcore_barrier()

    def edge_body(j, _):
        @pl.when(j >= 4)
        def _():
            pltpu.make_async_copy(
                ones.at[pl.ds(0, 50)], deg_sh.at[dbuf.at[j]], dsem).wait()

        pltpu.async_copy(
            ones.at[pl.ds(0, 50)], deg_sh.at[dbuf.at[j]], dsem, add=True)
        return 0
    lax.fori_loop(0, 200, edge_body, 0)
    for _ in range(4):
        pltpu.make_async_copy(
            ones.at[pl.ds(0, 50)], deg_sh.at[dbuf.at[0]], dsem).wait()

    @pl.when(c == 0)
    def _():
        for k in range(8):
            ch = s + 16 * k

            @pl.when(ch < NPC)
            def _():
                pltpu.sync_copy(ones.at[pl.ds(0, PC)],
                                cnt_sh.at[bbuf.at[k]], add=True)

    plsc.subcore_barrier()

    for k in range(8):
        ch = s + 16 * k

        @pl.when(ch < NPC)
        def _():
            sl = pl.ds(ch * PC, PC)
            pltpu.sync_copy(deg_sh.at[sl], zbuf)
            pltpu.sync_copy(zbuf, deg_out.at[pl.ds(c * N + ch * PC, PC)])

    @pl.when(jnp.logical_and(c == 0, s == 0))
    def _():
        pltpu.sync_copy(cnt_sh, zbuf.at[pl.ds(0, G)])
        pltpu.sync_copy(zbuf.at[pl.ds(0, G)], cnt_out)


CE = 128
CPT = 79
JUNK_ROWS = 3584


@functools.partial(
    pl.kernel,
    out_type=jax.ShapeDtypeStruct((2 * N, D), jnp.float32),
    mesh=_MESH,
    scratch_types=[
        pltpu.VMEM((80, CE), jnp.int32),
        pltpu.VMEM((8, CE // 4), jnp.int32),
        pltpu.VMEM((8, CE // 4), jnp.int32),
        pltpu.VMEM((CE, D), jnp.float32),
        pltpu.VMEM((CE, D), jnp.float32),
        pltpu.SemaphoreType.DMA,
        pltpu.SemaphoreType.DMA,
        pltpu.VMEM_SHARED((N, D), jnp.float32),
    ],
)
def _sc_edge(y, pk2d, s_out,
             ipack, sidx, didx, rb0, rb1, gsem, ssem, s_sh):
    c = lax.axis_index("c")
    s = lax.axis_index("s")
    wid = c * 16 + s
    stage = rb0.at[pl.ds(0, PC)]

    for k in range(8):
        ch = s + 16 * k

        @pl.when(ch < NPC)
        def _():
            sl = pl.ds(ch * PC, PC)
            pltpu.sync_copy(y.at[sl], stage)
            pltpu.sync_copy(stage, s_sh.at[sl])

    pltpu.sync_copy(pk2d.at[pl.ds(wid * 80, 80)], ipack)

    plsc.subcore_barrier()

    HC = CE // 4
    NH = 4 * CPT
    slots = [rb0.at[pl.ds(0, HC)], rb0.at[pl.ds(HC, HC)],
             rb0.at[pl.ds(2 * HC, HC)], rb0.at[pl.ds(3 * HC, HC)],
             rb1.at[pl.ds(0, HC)], rb1.at[pl.ds(HC, HC)],
             rb1.at[pl.ds(2 * HC, HC)], rb1.at[pl.ds(3 * HC, HC)]]

    def unpack(j, b):
        row = j // 4
        cb = HC * (b % 4)
        for v in range(2):
            pk = ipack[row, pl.ds(cb + 16 * v, 16)]
            sidx[b, pl.ds(16 * v, 16)] = jnp.bitwise_and(pk, 0xFFFF)
            didx[b, pl.ds(16 * v, 16)] = lax.shift_right_logical(pk, 16)

    def gather(b):
        pltpu.async_copy(y.at[sidx.at[b]], slots[b], gsem)

    def wait_gather(b):
        pltpu.make_async_copy(y.at[sidx.at[b]], slots[b], gsem).wait()

    def scatter(b):
        pltpu.async_copy(slots[b], s_sh.at[didx.at[b]], ssem, add=True)

    def wait_scatter(b):
        pltpu.make_async_copy(slots[b], s_sh.at[didx.at[b]], ssem).wait()

    for b in range(4):
        unpack(b, b)
        gather(b)

    def edge_body(i, _):
        for b in range(8):
            j = 8 * i + b
            wait_gather(b)
            scatter(b)

            @pl.when(j >= 4)
            def _():
                wait_scatter((b + 4) % 8)

            @pl.when(j + 4 < NH)
            def _():
                unpack(j + 4, (b + 4) % 8)
                gather((b + 4) % 8)
        return 0
    lax.fori_loop(0, NH // 8, edge_body, 0)

    for b in range(4):
        wait_gather(b)
        scatter(b)
        wait_scatter((b + 4) % 8)
    for b in range(4):
        wait_scatter(b)

    plsc.subcore_barrier()

    for k in range(8):
        ch = s + 16 * k

        @pl.when(ch < NPC)
        def _():
            sl = pl.ds(ch * PC, PC)
            pltpu.sync_copy(s_sh.at[sl], stage)
            pltpu.sync_copy(stage, s_out.at[pl.ds(c * N + ch * PC, PC)])


@functools.partial(
    pl.kernel,
    out_type=jax.ShapeDtypeStruct((2 * G, D), jnp.float32),
    mesh=_MESH,
    scratch_types=[
        pltpu.VMEM((4, PC), jnp.int32),
        pltpu.VMEM((PC, D), jnp.float32),
        pltpu.VMEM_SHARED((G, D), jnp.float32),
    ],
)
def _sc_pool(z, batch2d, zeros, p_out, bbuf, rbuf, p_sh):
    c = lax.axis_index("c")
    s = lax.axis_index("s")
    wid = c * 16 + s

    @pl.when(s == 0)
    def _():
        pltpu.sync_copy(zeros.at[pl.ds(0, G)], rbuf.at[pl.ds(0, G)])
        pltpu.sync_copy(rbuf.at[pl.ds(0, G)], p_sh)

    for k in range(4):
        ch = wid + 32 * k

        @pl.when(ch < NPC)
        def _():
            pltpu.sync_copy(batch2d.at[ch], bbuf.at[k])

    plsc.subcore_barrier()

    for k in range(4):
        ch = wid + 32 * k

        @pl.when(ch < NPC)
        def _():
            pltpu.sync_copy(z.at[pl.ds(ch * PC, PC)], rbuf)
            pltpu.sync_copy(rbuf, p_sh.at[bbuf.at[k]], add=True)

    plsc.subcore_barrier()

    @pl.when(s == 0)
    def _():
        pltpu.sync_copy(p_sh, rbuf.at[pl.ds(0, G)])
        pltpu.sync_copy(rbuf.at[pl.ds(0, G)], p_out.at[pl.ds(c * G, G)])


R = 1000


def _mm1_body(deg_ref, x_ref, w_ref, y_ref, dinv_ref):
    deg = deg_ref[0][:, :1] + deg_ref[1][:, :1] + 1.0
    dinv = lax.rsqrt(deg)
    dinv_ref[...] = dinv
    y_ref[...] = dinv * jnp.dot(
        x_ref[...], w_ref[...], preferred_element_type=jnp.float32)


def _tc_mm1(deg2, x, w):
    return pl.pallas_call(
        _mm1_body,
        grid=(N // R,),
        in_specs=[
            pl.BlockSpec((2, R, D), lambda r: (0, r, 0)),
            pl.BlockSpec((R, D), lambda r: (r, 0)),
            pl.BlockSpec((D, D), lambda r: (0, 0)),
        ],
        out_specs=[
            pl.BlockSpec((R, D), lambda r: (r, 0)),
            pl.BlockSpec((R, 1), lambda r: (r, 0)),
        ],
        out_shape=[
            jax.ShapeDtypeStruct((N, D), jnp.float32),
            jax.ShapeDtypeStruct((N, 1), jnp.float32),
        ],
    )(deg2, x, w)


def _junk_mask(r):
    rows = r * R + lax.broadcasted_iota(jnp.int32, (R, 1), 0)
    return jnp.where(rows < JUNK_ROWS, 2.0, 1.0)


def _layer_body(sp_ref, y_ref, dinv_ref, b_ref, w_ref, out_ref):
    h = sp_ref[0] + sp_ref[1] - _junk_mask(pl.program_id(0)) * y_ref[...]
    h = jnp.maximum(dinv_ref[...] * h + b_ref[...], 0.0)
    out_ref[...] = dinv_ref[...] * jnp.dot(
        h, w_ref[...], preferred_element_type=jnp.float32)


def _tc_layer(s_part, y, dinv, b, w):
    return pl.pallas_call(
        _layer_body,
        grid=(N // R,),
        in_specs=[
            pl.BlockSpec((2, R, D), lambda r: (0, r, 0)),
            pl.BlockSpec((R, D), lambda r: (r, 0)),
            pl.BlockSpec((R, 1), lambda r: (r, 0)),
            pl.BlockSpec((1, D), lambda r: (0, 0)),
            pl.BlockSpec((D, D), lambda r: (0, 0)),
        ],
        out_specs=pl.BlockSpec((R, D), lambda r: (r, 0)),
        out_shape=jax.ShapeDtypeStruct((N, D), jnp.float32),
    )(s_part, y, dinv, b, w)


def _z_body(sp_ref, y_ref, dinv_ref, b_ref, out_ref):
    s = sp_ref[0] + sp_ref[1] - _junk_mask(pl.program_id(0)) * y_ref[...]
    out_ref[...] = dinv_ref[...] * s + b_ref[...]


def _tc_z(s_part, y, dinv, b):
    return pl.pallas_call(
        _z_body,
        grid=(N // R,),
        in_specs=[
            pl.BlockSpec((2, R, D), lambda r: (0, r, 0)),
            pl.BlockSpec((R, D), lambda r: (r, 0)),
            pl.BlockSpec((R, 1), lambda r: (r, 0)),
            pl.BlockSpec((1, D), lambda r: (0, 0)),
        ],
        out_specs=pl.BlockSpec((R, D), lambda r: (r, 0)),
        out_shape=jax.ShapeDtypeStruct((N, D), jnp.float32),
    )(s_part, y, dinv, b)


def _final_body(pp_ref, cnt_ref, wl_ref, bl_ref, out_ref):
    pooled = (pp_ref[0] + pp_ref[1]) / jnp.maximum(cnt_ref[:, :1], 1.0)
    out_ref[...] = jnp.dot(
        pooled, wl_ref[...], preferred_element_type=jnp.float32) + bl_ref[...]


def _tc_final(pool_part, cnt16, wlin, blin):
    return pl.pallas_call(
        _final_body,
        out_shape=jax.ShapeDtypeStruct((G, wlin.shape[1]), jnp.float32),
    )(pool_part, cnt16, wlin, blin)


def kernel(x, edge_index, batch, W1, b1, W2, b2, W3, b3, Wlin, blin):
    dst2d = edge_index[1].reshape(E // 50, 50)
    batch2d = batch.reshape(NPC, PC)
    zeros = jnp.zeros((N, D), jnp.float32)

    sr = edge_index[0].reshape(32, E // 32)
    dr = edge_index[1].reshape(32, E // 32)
    jr = (112 * jnp.arange(32, dtype=jnp.int32)[:, None]
          + jnp.arange(112, dtype=jnp.int32)[None, :])
    zr = jnp.zeros((32, CE), jnp.int32)
    srcp = jnp.concatenate([sr, jr, zr], axis=1)
    dstp = jnp.concatenate([dr, jr, zr], axis=1)
    pk2d = jnp.bitwise_or(srcp, dstp << 16).reshape(32 * 80, CE)

    ones_in = jnp.ones((PC, D), jnp.float32)
    deg2, cnt = _sc_precompute(dst2d, batch2d, ones_in, zeros)

    y1, dinv = _tc_mm1(deg2.reshape(2, N, D), x, W1)
    s1 = _sc_edge(y1, pk2d).reshape(2, N, D)
    y2 = _tc_layer(s1, y1, dinv, b1.reshape(1, D), W2)
    s2 = _sc_edge(y2, pk2d).reshape(2, N, D)
    y3 = _tc_layer(s2, y2, dinv, b2.reshape(1, D), W3)
    s3 = _sc_edge(y3, pk2d).reshape(2, N, D)

    z = _tc_z(s3, y3, dinv, b3.reshape(1, D))
    pool_part = _sc_pool(z, batch2d, zeros).reshape(2, G, D)
    return _tc_final(pool_part, cnt, Wlin, blin.reshape(1, blin.shape[0]))
